# baseline (speedup 1.0000x reference)
import functools

import jax
import jax.numpy as jnp
from jax import lax
from jax.experimental import pallas as pl
from jax.experimental.pallas import tpu as pltpu
from jax.experimental.pallas import tpu_sc as plsc

N = 10000
E = 320000
D = 128

NC = 2
NS = 16
CPC = D // NC
CPT = 4
NW = N * CPT

SE = 3200
NSC = E // SE
NG = SE // 16


def _mm_body(feat_ref, cu_ref, w_ref, out_ref):
    x = feat_ref[...] * cu_ref[...]
    out_ref[0] = jnp.dot(x, w_ref[0], preferred_element_type=jnp.float32)


def _matmul(feat, cu, W):
    BLK = 2000
    w2 = W.reshape(D, NC, CPC).transpose(1, 0, 2)
    return pl.pallas_call(
        _mm_body,
        grid=(NC, N // BLK),
        in_specs=[
            pl.BlockSpec((BLK, D), lambda c, i: (i, 0)),
            pl.BlockSpec((BLK, 1), lambda c, i: (i, 0)),
            pl.BlockSpec((1, D, CPC), lambda c, i: (c, 0, 0)),
        ],
        out_specs=pl.BlockSpec((1, BLK, CPC), lambda c, i: (c, i, 0)),
        out_shape=jax.ShapeDtypeStruct((NC, N, CPC), jnp.float32),
    )(feat, cu, w2)


def _sc_body(h_hbm, src_hbm, dst_hbm, w_hbm, cv_hbm, out_hbm,
             table_v, acc_v, cv_v,
             src0, src1, dst0, dst1, w0, w1,
             sem_e0, sem_e1):
    c = lax.axis_index("c")
    s = lax.axis_index("s")
    tid = s * NC + c

    srcs = (src0, src1)
    dsts = (dst0, dst1)
    ws = (w0, w1)
    sem_e = (sem_e0, sem_e1)

    def chunk_off(sb):
        return lax.rem(sb + 3 * tid, NSC) * SE

    def stage_start(sb, slot):
        off = chunk_off(sb)
        pltpu.async_copy(src_hbm.at[pl.ds(off, SE)], srcs[slot], sem_e[slot])
        pltpu.async_copy(dst_hbm.at[pl.ds(off, SE)], dsts[slot], sem_e[slot])
        pltpu.async_copy(w_hbm.at[pl.ds(off, SE)], ws[slot], sem_e[slot])

    def stage_wait(slot):
        pltpu.make_async_copy(src_hbm.at[pl.ds(0, SE)], srcs[slot],
                              sem_e[slot]).wait()
        pltpu.make_async_copy(dst_hbm.at[pl.ds(0, SE)], dsts[slot],
                              sem_e[slot]).wait()
        pltpu.make_async_copy(w_hbm.at[pl.ds(0, SE)], ws[slot],
                              sem_e[slot]).wait()

    pltpu.sync_copy(h_hbm.at[c, s], table_v)
    pltpu.sync_copy(cv_hbm, cv_v)
    zero16 = jnp.zeros((16,), jnp.float32)

    def zero_body(i, _):
        acc_v[pl.ds(i * 16, 16)] = zero16
        return 0
    lax.fori_loop(0, NW // 16, zero_body, 0)

    stage_start(0, 0)

    def group_maker(sv, dv, wv):
        def group_body(g, _):
            base = g * 16
            src16 = sv[pl.ds(base, 16)]
            dst16 = dv[pl.ds(base, 16)]
            w16 = wv[pl.ds(base, 16)]
            a_s = src16 * CPT
            a_d = dst16 * CPT
            for r in range(CPT):
                ar = a_s + r if r else a_s
                dr = a_d + r if r else a_d
                v = plsc.load_gather(table_v, [ar])
                plsc.addupdate_scatter(acc_v, [dr], v * w16)
            return 0
        return group_body

    def pair_body(p, _):
        for b in range(2):
            sb = p * 2 + b
            stage_wait(b)
            if b == 0:
                stage_start(sb + 1, 1)
            else:
                @pl.when(sb < NSC - 1)
                def _():
                    stage_start(sb + 1, 0)
            lax.fori_loop(0, NG, group_maker(srcs[b], dsts[b], ws[b]), 0)
        return 0
    lax.fori_loop(0, NSC // 2, pair_body, 0)

    q4 = lax.shift_right_logical(lax.iota(jnp.int32, 16), 2)

    def cvm_body(g, _):
        idx = q4 + g * 4
        cvv = plsc.load_gather(cv_v, [idx])
        sl = pl.ds(g * 16, 16)
        acc_v[sl] = acc_v[sl] * cvv
        return 0
    lax.fori_loop(0, NW // 16, cvm_body, 0)
    pltpu.sync_copy(acc_v, out_hbm.at[c, s])


@jax.jit
def kernel(feat, W, cu, cv, edge_w, edge_index):
    h2 = _matmul(feat, cu, W)
    h3 = h2.reshape(NC, N, NS, CPT).transpose(0, 2, 1, 3).reshape(NC, NS, NW)

    src = edge_index[0]
    dst = edge_index[1]
    ew = edge_w.reshape(E)
    cv1 = cv.reshape(N)

    mesh = plsc.VectorSubcoreMesh(core_axis_name="c", subcore_axis_name="s")
    sc_fn = pl.kernel(
        _sc_body,
        out_type=jax.ShapeDtypeStruct((NC, NS, NW), jnp.float32),
        mesh=mesh,
        compiler_params=pltpu.CompilerParams(
            use_tc_tiling_on_sc=False, needs_layout_passes=False),
        scratch_types=[
            pltpu.VMEM((NW,), jnp.float32),
            pltpu.VMEM((NW,), jnp.float32),
            pltpu.VMEM((N,), jnp.float32),
            pltpu.VMEM((SE,), jnp.int32),
            pltpu.VMEM((SE,), jnp.int32),
            pltpu.VMEM((SE,), jnp.int32),
            pltpu.VMEM((SE,), jnp.int32),
            pltpu.VMEM((SE,), jnp.float32),
            pltpu.VMEM((SE,), jnp.float32),
            pltpu.SemaphoreType.DMA,
            pltpu.SemaphoreType.DMA,
        ],
    )
    out3 = sc_fn(h3, src, dst, ew, cv1)
    return out3.reshape(NC, NS, N, CPT).transpose(2, 0, 1, 3).reshape(N, D)


# speedup vs baseline: 4.4731x; 4.4731x over previous
import functools

import jax
import jax.numpy as jnp
import numpy as np
from jax import lax
from jax.experimental import pallas as pl
from jax.experimental.pallas import tpu as pltpu
from jax.experimental.pallas import tpu_sc as plsc

N = 10000
E = 320000
D = 128

NC = 2
NS = 16
CPC = D // NC
NQ = CPC // 16

CW = 128
ROWS_T = 160
EP = NS * ROWS_T * CW
SB = 80
NSB = ROWS_T // SB

RPW = 640
RPW_LAST = N - (NS - 1) * RPW
NB = 80

_PB = np.arange(32).reshape(2, 16).T.reshape(-1)
_PERM = np.concatenate(
    [c * CPC + q * 32 + _PB for c in range(NC) for q in range(CPC // 32)])


def _mm_body(feat_ref, cu_ref, w_ref, out_ref):
    x = feat_ref[...] * cu_ref[...]
    h = jnp.dot(x, w_ref[0], preferred_element_type=jnp.float32)
    out_ref[0] = h.astype(jnp.bfloat16)


def _matmul(feat, cu, W):
    BLK = 2000
    w2 = W[:, _PERM].reshape(D, NC, CPC).transpose(1, 0, 2)
    return pl.pallas_call(
        _mm_body,
        grid=(NC, N // BLK),
        in_specs=[
            pl.BlockSpec((BLK, D), lambda c, i: (i, 0)),
            pl.BlockSpec((BLK, 1), lambda c, i: (i, 0)),
            pl.BlockSpec((1, D, CPC), lambda c, i: (c, 0, 0)),
        ],
        out_specs=pl.BlockSpec((1, BLK, CPC), lambda c, i: (c, i, 0)),
        out_shape=jax.ShapeDtypeStruct((NC, N, CPC), jnp.bfloat16),
    )(feat, cu, w2)


def _scale_chunk(w_ref, j, in_ref, out_ref):
    for g in range(CW // 16):
        wvec = w_ref[j, pl.ds(g * 16, 16)]
        for qq in range(0, 16, 4):
            vals = []
            for q in range(qq, qq + 4):
                wq = wvec[q]
                i = g * 16 + q
                for u in range(CPC // 32):
                    x32 = in_ref[i, pl.ds(u * 32, 32)]
                    lo, hi = plsc.unpack(x32, format=plsc.PackFormat.INTERLEAVED)
                    vals.append(lo * wq)
                    vals.append(hi * wq)
            k = 0
            for q in range(qq, qq + 4):
                i = g * 16 + q
                for r in range(NQ):
                    out_ref[i, pl.ds(r * 16, 16)] = vals[k]
                    k += 1


def _sc_body(ha_hbm, hb_hbm, src_hbm, dst_hbm, w_hbm, cv_hbm, out_hbm,
             acc_sh, src_v, dst_v, w_v,
             rows0, rows1, mrows0, mrows1, node_v, cv_v,
             sem_g0, sem_g1, sem_s0, sem_s1):
    c = lax.axis_index("c")
    s = lax.axis_index("s")
    r0 = s * RPW
    nrows = jnp.where(s < NS - 1, RPW, RPW_LAST)
    nblk = jnp.where(s < NS - 1, RPW // NB, RPW_LAST // NB)

    rows = (rows0, rows1)
    mrows = (mrows0, mrows1)
    sem_g = (sem_g0, sem_g1)
    sem_s = (sem_s0, sem_s1)

    def gather_start(jj, b):
        @pl.when(c == 0)
        def _():
            pltpu.async_copy(ha_hbm.at[src_v.at[jj]], rows[b], sem_g[b])
        @pl.when(c == 1)
        def _():
            pltpu.async_copy(hb_hbm.at[src_v.at[jj]], rows[b], sem_g[b])

    def gather_wait(jj, b):
        @pl.when(c == 0)
        def _():
            pltpu.make_async_copy(
                ha_hbm.at[src_v.at[jj]], rows[b], sem_g[b]).wait()
        @pl.when(c == 1)
        def _():
            pltpu.make_async_copy(
                hb_hbm.at[src_v.at[jj]], rows[b], sem_g[b]).wait()

    zero16 = jnp.zeros((16,), jnp.float32)

    def zero_body(i, _):
        for q in range(NQ):
            node_v[i, pl.ds(q * 16, 16)] = zero16
        return 0
    lax.fori_loop(0, NB, zero_body, 0)

    def zcopy_body(b, _):
        pltpu.sync_copy(node_v, acc_sh.at[pl.ds(r0 + b * NB, NB)])
        return 0
    lax.fori_loop(0, nblk, zcopy_body, 0)

    plsc.subcore_barrier()

    def sb_body(sb, _):
        pltpu.sync_copy(src_hbm.at[s, pl.ds(sb * SB, SB)], src_v)
        pltpu.sync_copy(dst_hbm.at[s, pl.ds(sb * SB, SB)], dst_v)
        pltpu.sync_copy(w_hbm.at[s, pl.ds(sb * SB, SB)], w_v)

        gather_start(0, 0)

        def pair_body(j0, _):
            for b in range(2):
                j = j0 + b
                gather_wait(j, b)
                if b == 0:
                    gather_start(j + 1, 1)
                else:
                    @pl.when(j0 < SB - 2)
                    def _():
                        gather_start(j + 1, 0)
                @pl.when(j0 >= 2)
                def _():
                    pltpu.make_async_copy(
                        mrows[b], acc_sh.at[dst_v.at[j]], sem_s[b]).wait()
                _scale_chunk(w_v, j, rows[b], mrows[b])
                pltpu.async_copy(
                    mrows[b], acc_sh.at[dst_v.at[j]], sem_s[b], add=True)
            return 0
        lax.fori_loop(0, SB // 2, lambda p, _: pair_body(p * 2, _), 0)

        for b in range(2):
            pltpu.make_async_copy(
                mrows[b], acc_sh.at[dst_v.at[SB - 2 + b]], sem_s[b]).wait()
        return 0
    lax.fori_loop(0, NSB, sb_body, 0)

    plsc.subcore_barrier()

    def out_blk_body(b, _):
        rb = r0 + b * NB
        pltpu.sync_copy(acc_sh.at[pl.ds(rb, NB)], node_v)
        pltpu.sync_copy(cv_hbm.at[pl.ds(rb, NB)], cv_v)

        def out_body(g, _):
            cvec = cv_v[pl.ds(g * 16, 16)]
            for qq in range(0, 16, 4):
                vals = []
                for q in range(qq, qq + 4):
                    cq = cvec[q]
                    i = g * 16 + q
                    for r in range(NQ):
                        vals.append(node_v[i, pl.ds(r * 16, 16)] * cq)
                k = 0
                for q in range(qq, qq + 4):
                    i = g * 16 + q
                    for r in range(NQ):
                        node_v[i, pl.ds(r * 16, 16)] = vals[k]
                        k += 1
            return 0
        lax.fori_loop(0, NB // 16, out_body, 0)
        pltpu.sync_copy(node_v, out_hbm.at[pl.ds(rb, NB), pl.ds(c * CPC, CPC)])
        return 0
    lax.fori_loop(0, nblk, out_blk_body, 0)


@jax.jit
def kernel(feat, W, cu, cv, edge_w, edge_index):
    h = _matmul(feat, cu, W)
    ha = h[0]
    hb = h[1]

    pad = EP - E
    pad_idx = (jnp.arange(pad, dtype=jnp.int32) * 37) % N
    src = jnp.concatenate([edge_index[0], pad_idx]).reshape(NS, ROWS_T, CW)
    dst = jnp.concatenate([edge_index[1], pad_idx]).reshape(NS, ROWS_T, CW)
    ew = jnp.concatenate(
        [edge_w.reshape(E), jnp.zeros((pad,), jnp.float32)]
    ).reshape(NS, ROWS_T, CW)
    cv1 = cv.reshape(N)

    mesh = plsc.VectorSubcoreMesh(core_axis_name="c", subcore_axis_name="s")
    sc_fn = pl.kernel(
        _sc_body,
        out_type=jax.ShapeDtypeStruct((N, D), jnp.float32),
        mesh=mesh,
        compiler_params=pltpu.CompilerParams(
            use_tc_tiling_on_sc=False, needs_layout_passes=False),
        scratch_types=[
            pltpu.VMEM_SHARED((N, CPC), jnp.float32),
            pltpu.VMEM((SB, CW), jnp.int32),
            pltpu.VMEM((SB, CW), jnp.int32),
            pltpu.VMEM((SB, CW), jnp.float32),
            pltpu.VMEM((CW, CPC), jnp.bfloat16),
            pltpu.VMEM((CW, CPC), jnp.bfloat16),
            pltpu.VMEM((CW, CPC), jnp.float32),
            pltpu.VMEM((CW, CPC), jnp.float32),
            pltpu.VMEM((NB, CPC), jnp.float32),
            pltpu.VMEM((NB,), jnp.float32),
            pltpu.SemaphoreType.DMA,
            pltpu.SemaphoreType.DMA,
            pltpu.SemaphoreType.DMA,
            pltpu.SemaphoreType.DMA,
        ],
    )
    return sc_fn(ha, hb, src, dst, ew, cv1)


# speedup vs baseline: 4.7169x; 1.0545x over previous
import functools

import jax
import jax.numpy as jnp
import numpy as np
from jax import lax
from jax.experimental import pallas as pl
from jax.experimental.pallas import tpu as pltpu
from jax.experimental.pallas import tpu_sc as plsc

N = 10000
E = 320000
D = 128

NC = 2
NS = 16
CPC = D // NC
NQ = CPC // 16

CW = 128
ROWS_T = 160
EP = NS * ROWS_T * CW
SB = 80
NSB = ROWS_T // SB

RPW = 640
RPW_LAST = N - (NS - 1) * RPW
NB = 80

_PB = np.arange(32).reshape(2, 16).T.reshape(-1)
_PERM = np.concatenate(
    [c * CPC + q * 32 + _PB for c in range(NC) for q in range(CPC // 32)])


def _mm_body(feat_ref, cu_ref, w_ref, out_ref):
    x = feat_ref[...] * cu_ref[...]
    for c in range(NC):
        h = jnp.dot(x, w_ref[c], preferred_element_type=jnp.float32)
        out_ref[c] = h.astype(jnp.bfloat16)


def _matmul(feat, cu, W):
    BLK = 2000
    w2 = W[:, _PERM].reshape(D, NC, CPC).transpose(1, 0, 2)
    return pl.pallas_call(
        _mm_body,
        grid=(N // BLK,),
        in_specs=[
            pl.BlockSpec((BLK, D), lambda i: (i, 0)),
            pl.BlockSpec((BLK, 1), lambda i: (i, 0)),
            pl.BlockSpec((NC, D, CPC), lambda i: (0, 0, 0)),
        ],
        out_specs=pl.BlockSpec((NC, BLK, CPC), lambda i: (0, i, 0)),
        out_shape=jax.ShapeDtypeStruct((NC, N, CPC), jnp.bfloat16),
    )(feat, cu, w2)


def _scale_chunk(w_ref, j, in_ref, out_ref):
    for g in range(CW // 16):
        wvec = w_ref[j, pl.ds(g * 16, 16)]
        for qq in range(0, 16, 4):
            vals = []
            for q in range(qq, qq + 4):
                wq = wvec[q]
                i = g * 16 + q
                for u in range(CPC // 32):
                    x32 = in_ref[i, pl.ds(u * 32, 32)]
                    lo, hi = plsc.unpack(x32, format=plsc.PackFormat.INTERLEAVED)
                    vals.append(lo * wq)
                    vals.append(hi * wq)
            k = 0
            for q in range(qq, qq + 4):
                i = g * 16 + q
                for r in range(NQ):
                    out_ref[i, pl.ds(r * 16, 16)] = vals[k]
                    k += 1


def _sc_body(ha_hbm, hb_hbm, ei_hbm, w_hbm, cv_hbm, out_hbm,
             acc_sh, src_v, dst_v, w_v,
             rows0, rows1, mrows0, mrows1, node_v, cv_v,
             sem_g0, sem_g1, sem_s0, sem_s1):
    c = lax.axis_index("c")
    s = lax.axis_index("s")
    r0 = s * RPW
    nrows = jnp.where(s < NS - 1, RPW, RPW_LAST)
    nblk = jnp.where(s < NS - 1, RPW // NB, RPW_LAST // NB)

    rows = (rows0, rows1)
    mrows = (mrows0, mrows1)
    sem_g = (sem_g0, sem_g1)
    sem_s = (sem_s0, sem_s1)

    def gather_start(jj, b):
        @pl.when(c == 0)
        def _():
            pltpu.async_copy(ha_hbm.at[src_v.at[jj]], rows[b], sem_g[b])
        @pl.when(c == 1)
        def _():
            pltpu.async_copy(hb_hbm.at[src_v.at[jj]], rows[b], sem_g[b])

    def gather_wait(jj, b):
        @pl.when(c == 0)
        def _():
            pltpu.make_async_copy(
                ha_hbm.at[src_v.at[jj]], rows[b], sem_g[b]).wait()
        @pl.when(c == 1)
        def _():
            pltpu.make_async_copy(
                hb_hbm.at[src_v.at[jj]], rows[b], sem_g[b]).wait()

    zero16 = jnp.zeros((16,), jnp.float32)

    def zero_body(i, _):
        for q in range(NQ):
            node_v[i, pl.ds(q * 16, 16)] = zero16
        return 0
    lax.fori_loop(0, NB, zero_body, 0)

    def zcopy_body(b, _):
        pltpu.sync_copy(node_v, acc_sh.at[pl.ds(r0 + b * NB, NB)])
        return 0
    lax.fori_loop(0, nblk, zcopy_body, 0)

    plsc.subcore_barrier()

    def sb_body(sb, _):
        pltpu.sync_copy(ei_hbm.at[0, s, pl.ds(sb * SB, SB)], src_v)
        pltpu.sync_copy(ei_hbm.at[1, s, pl.ds(sb * SB, SB)], dst_v)
        pltpu.sync_copy(w_hbm.at[s, pl.ds(sb * SB, SB)], w_v)

        gather_start(0, 0)

        def pair_body(j0, _):
            for b in range(2):
                j = j0 + b
                gather_wait(j, b)
                if b == 0:
                    gather_start(j + 1, 1)
                else:
                    @pl.when(j0 < SB - 2)
                    def _():
                        gather_start(j + 1, 0)
                @pl.when(j0 >= 2)
                def _():
                    pltpu.make_async_copy(
                        mrows[b], acc_sh.at[dst_v.at[j]], sem_s[b]).wait()
                _scale_chunk(w_v, j, rows[b], mrows[b])
                pltpu.async_copy(
                    mrows[b], acc_sh.at[dst_v.at[j]], sem_s[b], add=True)
            return 0
        lax.fori_loop(0, SB // 2, lambda p, _: pair_body(p * 2, _), 0)

        for b in range(2):
            pltpu.make_async_copy(
                mrows[b], acc_sh.at[dst_v.at[SB - 2 + b]], sem_s[b]).wait()
        return 0
    lax.fori_loop(0, NSB, sb_body, 0)

    plsc.subcore_barrier()

    def out_blk_body(b, _):
        rb = r0 + b * NB
        pltpu.sync_copy(acc_sh.at[pl.ds(rb, NB)], node_v)
        pltpu.sync_copy(cv_hbm.at[0, pl.ds(rb, NB)], cv_v)

        def out_body(g, _):
            cvec = cv_v[pl.ds(g * 16, 16)]
            for qq in range(0, 16, 4):
                vals = []
                for q in range(qq, qq + 4):
                    cq = cvec[q]
                    i = g * 16 + q
                    for r in range(NQ):
                        vals.append(node_v[i, pl.ds(r * 16, 16)] * cq)
                k = 0
                for q in range(qq, qq + 4):
                    i = g * 16 + q
                    for r in range(NQ):
                        node_v[i, pl.ds(r * 16, 16)] = vals[k]
                        k += 1
            return 0
        lax.fori_loop(0, NB // 16, out_body, 0)
        pltpu.sync_copy(node_v, out_hbm.at[pl.ds(rb, NB), pl.ds(c * CPC, CPC)])
        return 0
    lax.fori_loop(0, nblk, out_blk_body, 0)


@jax.jit
def kernel(feat, W, cu, cv, edge_w, edge_index):
    h = _matmul(feat, cu, W)
    ha = h[0]
    hb = h[1]

    pad = EP - E
    pad_idx = (jnp.arange(pad, dtype=jnp.int32) * 37) % N
    ei = jnp.concatenate(
        [edge_index, jnp.broadcast_to(pad_idx, (2, pad))], axis=1
    ).reshape(2, NS, ROWS_T, CW)
    ew = jnp.concatenate(
        [edge_w.reshape(E), jnp.zeros((pad,), jnp.float32)]
    ).reshape(NS, ROWS_T, CW)
    cv1 = cv.reshape(1, N)

    mesh = plsc.VectorSubcoreMesh(core_axis_name="c", subcore_axis_name="s")
    sc_fn = pl.kernel(
        _sc_body,
        out_type=jax.ShapeDtypeStruct((N, D), jnp.float32),
        mesh=mesh,
        compiler_params=pltpu.CompilerParams(
            use_tc_tiling_on_sc=False, needs_layout_passes=False),
        scratch_types=[
            pltpu.VMEM_SHARED((N, CPC), jnp.float32),
            pltpu.VMEM((SB, CW), jnp.int32),
            pltpu.VMEM((SB, CW), jnp.int32),
            pltpu.VMEM((SB, CW), jnp.float32),
            pltpu.VMEM((CW, CPC), jnp.bfloat16),
            pltpu.VMEM((CW, CPC), jnp.bfloat16),
            pltpu.VMEM((CW, CPC), jnp.float32),
            pltpu.VMEM((CW, CPC), jnp.float32),
            pltpu.VMEM((NB, CPC), jnp.float32),
            pltpu.VMEM((NB,), jnp.float32),
            pltpu.SemaphoreType.DMA,
            pltpu.SemaphoreType.DMA,
            pltpu.SemaphoreType.DMA,
            pltpu.SemaphoreType.DMA,
        ],
    )
    return sc_fn(ha, hb, ei, ew, cv1)


# speedup vs baseline: 4.9721x; 1.0541x over previous
import functools

import jax
import jax.numpy as jnp
import numpy as np
from jax import lax
from jax.experimental import pallas as pl
from jax.experimental.pallas import tpu as pltpu
from jax.experimental.pallas import tpu_sc as plsc

N = 10000
E = 320000
D = 128

NC = 2
NS = 16
CPC = D // NC
NQ = CPC // 16

CW = 128
ROWS_T = 160
EP = NS * ROWS_T * CW
SB = 80
NSB = ROWS_T // SB

RPW = 640
RPW_LAST = N - (NS - 1) * RPW
NB = 80

_PB = np.arange(32).reshape(2, 16).T.reshape(-1)
_PERM = np.concatenate(
    [c * CPC + q * 32 + _PB for c in range(NC) for q in range(CPC // 32)])


def _mm_body(feat_ref, cu_ref, w_ref, out_ref):
    x = feat_ref[...] * cu_ref[...]
    for c in range(NC):
        h = jnp.dot(x, w_ref[c], preferred_element_type=jnp.float32)
        out_ref[c] = h.astype(jnp.bfloat16)


def _matmul(feat, cu, W):
    BLK = 2000
    w2 = W[:, _PERM].reshape(D, NC, CPC).transpose(1, 0, 2)
    return pl.pallas_call(
        _mm_body,
        grid=(N // BLK,),
        in_specs=[
            pl.BlockSpec((BLK, D), lambda i: (i, 0)),
            pl.BlockSpec((BLK, 1), lambda i: (i, 0)),
            pl.BlockSpec((NC, D, CPC), lambda i: (0, 0, 0)),
        ],
        out_specs=pl.BlockSpec((NC, BLK, CPC), lambda i: (0, i, 0)),
        out_shape=jax.ShapeDtypeStruct((NC, N, CPC), jnp.bfloat16),
    )(feat, cu, w2)


def _scale_chunk(w_ref, j, in_ref, out_ref):
    for g in range(CW // 16):
        wvec = w_ref[j, pl.ds(g * 16, 16)]
        for qq in range(0, 16, 4):
            vals = []
            for q in range(qq, qq + 4):
                wq = wvec[q]
                i = g * 16 + q
                for u in range(CPC // 32):
                    x32 = in_ref[i, pl.ds(u * 32, 32)]
                    lo, hi = plsc.unpack(x32, format=plsc.PackFormat.INTERLEAVED)
                    vals.append(lo * wq)
                    vals.append(hi * wq)
            k = 0
            for q in range(qq, qq + 4):
                i = g * 16 + q
                for r in range(NQ):
                    out_ref[i, pl.ds(r * 16, 16)] = vals[k]
                    k += 1


def _sc_body(ha_hbm, hb_hbm, ei_hbm, w_hbm, cv_hbm, out_hbm,
             acc_sh, src_v, dst_v, w_v,
             rows0, rows1, mrows0, mrows1, node_v, cv_v,
             sem_g0, sem_g1, sem_s0, sem_s1):
    c = lax.axis_index("c")
    s = lax.axis_index("s")
    r0 = s * RPW
    nrows = jnp.where(s < NS - 1, RPW, RPW_LAST)
    nblk = jnp.where(s < NS - 1, RPW // NB, RPW_LAST // NB)

    rows = (rows0, rows1)
    mrows = (mrows0, mrows1)
    sem_g = (sem_g0, sem_g1)
    sem_s = (sem_s0, sem_s1)

    def gather_start(jj, b):
        @pl.when(c == 0)
        def _():
            pltpu.async_copy(ha_hbm.at[src_v.at[jj]], rows[b], sem_g[b])
        @pl.when(c == 1)
        def _():
            pltpu.async_copy(hb_hbm.at[src_v.at[jj]], rows[b], sem_g[b])

    def gather_wait(jj, b):
        @pl.when(c == 0)
        def _():
            pltpu.make_async_copy(
                ha_hbm.at[src_v.at[jj]], rows[b], sem_g[b]).wait()
        @pl.when(c == 1)
        def _():
            pltpu.make_async_copy(
                hb_hbm.at[src_v.at[jj]], rows[b], sem_g[b]).wait()

    zero16 = jnp.zeros((16,), jnp.float32)

    def zero_body(i, _):
        for q in range(NQ):
            node_v[i, pl.ds(q * 16, 16)] = zero16
        return 0
    lax.fori_loop(0, NB, zero_body, 0)

    def zcopy_body(b, _):
        pltpu.sync_copy(node_v, acc_sh.at[pl.ds(r0 + b * NB, NB)])
        return 0
    lax.fori_loop(0, nblk, zcopy_body, 0)

    plsc.subcore_barrier()

    def sb_body(sb, _):
        pltpu.sync_copy(ei_hbm.at[0, s, pl.ds(sb * SB, SB)], src_v)
        pltpu.sync_copy(ei_hbm.at[1, s, pl.ds(sb * SB, SB)], dst_v)
        pltpu.sync_copy(w_hbm.at[s, pl.ds(sb * SB, SB)], w_v)

        gather_start(0, 0)

        def pair_body(j0, _):
            for b in range(2):
                j = j0 + b
                gather_wait(j, b)
                if b == 0:
                    gather_start(j + 1, 1)
                else:
                    @pl.when(j0 < SB - 2)
                    def _():
                        gather_start(j + 1, 0)
                @pl.when(j0 >= 2)
                def _():
                    pltpu.make_async_copy(
                        mrows[b], acc_sh.at[dst_v.at[j]], sem_s[b]).wait()
                _scale_chunk(w_v, j, rows[b], mrows[b])
                pltpu.async_copy(
                    mrows[b], acc_sh.at[dst_v.at[j]], sem_s[b], add=True)
            return 0
        lax.fori_loop(0, SB // 2, lambda p, _: pair_body(p * 2, _), 0)

        for b in range(2):
            pltpu.make_async_copy(
                mrows[b], acc_sh.at[dst_v.at[SB - 2 + b]], sem_s[b]).wait()
        return 0
    lax.fori_loop(0, NSB, sb_body, 0)

    plsc.subcore_barrier()

    def out_blk_body(b, _):
        rb = r0 + b * NB
        pltpu.sync_copy(acc_sh.at[pl.ds(rb, NB)], node_v)
        pltpu.sync_copy(cv_hbm.at[0, pl.ds(rb, NB)], cv_v)

        def out_body(g, _):
            cvec = cv_v[pl.ds(g * 16, 16)]
            for qq in range(0, 16, 4):
                vals = []
                for q in range(qq, qq + 4):
                    cq = cvec[q]
                    i = g * 16 + q
                    for r in range(NQ):
                        vals.append(node_v[i, pl.ds(r * 16, 16)] * cq)
                k = 0
                for q in range(qq, qq + 4):
                    i = g * 16 + q
                    for r in range(NQ):
                        node_v[i, pl.ds(r * 16, 16)] = vals[k]
                        k += 1
            return 0
        lax.fori_loop(0, NB // 16, out_body, 0)
        pltpu.sync_copy(node_v, out_hbm.at[pl.ds(rb, NB), pl.ds(c * CPC, CPC)])
        return 0
    lax.fori_loop(0, nblk, out_blk_body, 0)


@jax.jit
def kernel(feat, W, cu, cv, edge_w, edge_index):
    h = _matmul(feat, cu, W)
    ha = h[0]
    hb = h[1]

    pad = EP - E
    pad_idx = (jnp.arange(pad, dtype=jnp.int32) * 37) % N
    ei = jnp.concatenate(
        [edge_index, jnp.broadcast_to(pad_idx, (2, pad))], axis=1
    ).reshape(2, NS, ROWS_T, CW)
    ew = jnp.concatenate(
        [edge_w, jnp.zeros((pad, 1), jnp.float32)], axis=0
    ).reshape(NS, ROWS_T, CW)
    cv1 = cv.reshape(1, N)

    mesh = plsc.VectorSubcoreMesh(core_axis_name="c", subcore_axis_name="s")
    sc_fn = pl.kernel(
        _sc_body,
        out_type=jax.ShapeDtypeStruct((N, D), jnp.float32),
        mesh=mesh,
        compiler_params=pltpu.CompilerParams(
            use_tc_tiling_on_sc=False, needs_layout_passes=False),
        scratch_types=[
            pltpu.VMEM_SHARED((N, CPC), jnp.float32),
            pltpu.VMEM((SB, CW), jnp.int32),
            pltpu.VMEM((SB, CW), jnp.int32),
            pltpu.VMEM((SB, CW), jnp.float32),
            pltpu.VMEM((CW, CPC), jnp.bfloat16),
            pltpu.VMEM((CW, CPC), jnp.bfloat16),
            pltpu.VMEM((CW, CPC), jnp.float32),
            pltpu.VMEM((CW, CPC), jnp.float32),
            pltpu.VMEM((NB, CPC), jnp.float32),
            pltpu.VMEM((NB,), jnp.float32),
            pltpu.SemaphoreType.DMA,
            pltpu.SemaphoreType.DMA,
            pltpu.SemaphoreType.DMA,
            pltpu.SemaphoreType.DMA,
        ],
    )
    return sc_fn(ha, hb, ei, ew, cv1)
